# baseline (speedup 1.0000x reference)
import functools

import jax
import jax.numpy as jnp
from jax import lax
from jax.experimental import pallas as pl
from jax.experimental.pallas import tpu as pltpu
from jax.experimental.pallas import tpu_sc as plsc

_MAX_NORM = 1.0

_B = 4096
_NW = 32
_RW = _B // _NW
_R = 8
_NCH = _RW // _R

_FIELDS = (
    (12, 64, 0),
    (48, 64, 768),
    (24, 32, 3840),
    (12, 64, 4608),
    (12, 64, 5376),
    (23, 16, 6144),
)
_OUT_D = 6512


def _normalize_tables_tc(*tables):
    def body(*refs):
        n = len(refs) // 2
        for src, dst in zip(refs[:n], refs[n:]):
            x = src[...]
            nrm = jnp.sqrt(jnp.sum(x * x, axis=-1, keepdims=True))
            scale = jnp.where(nrm > _MAX_NORM,
                              _MAX_NORM / jnp.maximum(nrm, 1e-12), 1.0)
            dst[...] = x * scale

    out_shapes = [jax.ShapeDtypeStruct(t.shape, t.dtype) for t in tables]
    return pl.pallas_call(body, out_shape=out_shapes)(*tables)


def _sc_gather(tables, idxs):
    mesh = plsc.VectorSubcoreMesh(core_axis_name="c", subcore_axis_name="s")

    idx_scratch = [pltpu.VMEM((_RW * n,), jnp.int32) for n, _, _ in _FIELDS]
    buf_scratch = [pltpu.VMEM((_R * n, w), jnp.float32)
                   for n, w, _ in _FIELDS] * 2

    @functools.partial(
        pl.kernel,
        mesh=mesh,
        compiler_params=pltpu.CompilerParams(use_tc_tiling_on_sc=False),
        out_type=[jax.ShapeDtypeStruct((_B * n, w), jnp.float32)
                  for n, w, _ in _FIELDS],
        scratch_types=idx_scratch + buf_scratch + [
            pltpu.SemaphoreType.DMA, pltpu.SemaphoreType.DMA],
    )
    def k(*refs):
        tabs = refs[0:6]
        idxh = refs[6:12]
        outs = refs[12:18]
        idxv = refs[18:24]
        buf_a = refs[24:30]
        buf_b = refs[30:36]
        sem_g, sem_w = refs[36], refs[37]

        wid = lax.axis_index("s") * 2 + lax.axis_index("c")
        for h, v in zip(idxh, idxv):
            pltpu.sync_copy(h.at[wid], v)
        row0 = wid * _RW

        def fire_gathers(c, bset):
            for (n, w, _), tab, iv, buf in zip(_FIELDS, tabs, idxv, bset):
                pltpu.async_copy(tab.at[iv.at[pl.ds(c * n * _R, n * _R)]],
                                 buf, sem_g)

        def drain_gathers(bset):
            for (n, w, _), tab, buf in zip(_FIELDS, tabs, bset):
                pltpu.make_async_copy(tab.at[pl.ds(0, n * _R)], buf,
                                      sem_g).wait()

        def fire_writes(c, bset):
            for (n, w, _), o, buf in zip(_FIELDS, outs, bset):
                pltpu.async_copy(buf,
                                 o.at[pl.ds((row0 + c * _R) * n, n * _R)],
                                 sem_w)

        def drain_writes(bset):
            for (n, w, _), o, buf in zip(_FIELDS, outs, bset):
                pltpu.make_async_copy(buf, o.at[pl.ds(row0 * n, n * _R)],
                                      sem_w).wait()

        fire_gathers(0, buf_a)

        def pair(i, carry):
            ca = 2 * i
            drain_gathers(buf_a)

            @pl.when(i >= 1)
            def _():
                drain_writes(buf_b)

            fire_gathers(ca + 1, buf_b)
            fire_writes(ca, buf_a)
            drain_gathers(buf_b)

            @pl.when(i + 1 < _NCH // 2)
            def _():
                drain_writes(buf_a)
                fire_gathers(ca + 2, buf_a)

            fire_writes(ca + 1, buf_b)
            return carry

        lax.fori_loop(0, _NCH // 2, pair, 0)
        drain_writes(buf_a)
        drain_writes(buf_b)

    return k(*tables, *idxs)


def kernel(pokemon_state, move_state, type_state, ability_state, item_state,
           fieldeffect_state, pokemon_table, move_table, type_table,
           ability_table, item_table, fieldeffect_tables):
    tables = _normalize_tables_tc(
        pokemon_table, move_table, type_table, ability_table, item_table,
        fieldeffect_tables.reshape(46, 16))

    fe_idx = jnp.arange(23, dtype=jnp.int32) * 2 + fieldeffect_state
    idxs = [s.reshape(_NW, _RW * n) for s, (n, _, _) in zip(
        (pokemon_state, move_state, type_state, ability_state, item_state,
         fe_idx), _FIELDS)]

    outs = _sc_gather(tables, idxs)
    return jnp.concatenate(
        [o.reshape(_B, n * w) for o, (n, w, _) in zip(outs, _FIELDS)],
        axis=1)


# speedup vs baseline: 1.3203x; 1.3203x over previous
import functools

import jax
import jax.numpy as jnp
import numpy as np
from jax import lax
from jax.experimental import pallas as pl
from jax.experimental.pallas import tpu as pltpu
from jax.experimental.pallas import tpu_sc as plsc

_MAX_NORM = 1.0

_B = 4096
_SUBROWS = 407
_NW = 32
_GW = 1408
_NGROUPS = _SUBROWS * (_B // _NW) // _GW


def _qmaps():
    slot, scale, base = [], [], []
    for off, n, k, tb in ((0, 12, 4, 0), (12, 48, 4, 4096),
                          (60, 24, 2, 8192), (84, 12, 4, 8232),
                          (96, 12, 4, 9432)):
        for i in range(n):
            for j in range(k):
                slot.append(off + i)
                scale.append(k)
                base.append(tb + j)
    for j in range(23):
        slot.append(108 + j)
        scale.append(1)
        base.append(13528 + 2 * j)
    mk = lambda x: np.asarray(x, dtype=np.int32)
    return mk(slot), mk(scale), mk(base)


_QSLOT, _QSCALE, _QBASE = _qmaps()


def _normalize_tables_tc(*tables):
    def body(*refs):
        n = len(refs) // 2
        for src, dst in zip(refs[:n], refs[n:]):
            x = src[...]
            nrm = jnp.sqrt(jnp.sum(x * x, axis=-1, keepdims=True))
            scale = jnp.where(nrm > _MAX_NORM,
                              _MAX_NORM / jnp.maximum(nrm, 1e-12), 1.0)
            dst[...] = x * scale

    out_shapes = [jax.ShapeDtypeStruct(t.shape, t.dtype) for t in tables]
    return pl.pallas_call(body, out_shape=out_shapes)(*tables)


def _sc_gather(u, idx3d):
    mesh = plsc.VectorSubcoreMesh(core_axis_name="c", subcore_axis_name="s")

    gw = _GW

    @functools.partial(
        pl.kernel,
        mesh=mesh,
        compiler_params=pltpu.CompilerParams(use_tc_tiling_on_sc=False),
        out_type=jax.ShapeDtypeStruct((_B * _SUBROWS, 16), jnp.float32),
        scratch_types=[
            pltpu.VMEM((_NGROUPS, gw), jnp.int32),
            pltpu.VMEM((2 * gw, 16), jnp.float32),
            pltpu.SemaphoreType.DMA,
            pltpu.SemaphoreType.DMA,
        ],
    )
    def k(u_hbm, idx_hbm, out_hbm, idx_v, buf_v, sem_g, sem_w):
        wid = lax.axis_index("s") * 2 + lax.axis_index("c")
        pltpu.sync_copy(idx_hbm.at[wid], idx_v)
        out_w0 = wid * (_NGROUPS * gw)

        def fire(g, off):
            pltpu.async_copy(u_hbm.at[idx_v.at[g]],
                             buf_v.at[pl.ds(off, gw)], sem_g)

        fire(0, 0)

        def group(g, carry):
            off_cur = (g % 2) * gw
            off_next = ((g + 1) % 2) * gw

            @pl.when(g >= 1)
            def _():
                pltpu.make_async_copy(buf_v.at[pl.ds(off_next, gw)],
                                      out_hbm.at[pl.ds(out_w0, gw)],
                                      sem_w).wait()

            @pl.when(g + 1 < _NGROUPS)
            def _():
                fire(g + 1, off_next)

            pltpu.make_async_copy(u_hbm.at[pl.ds(0, gw)],
                                  buf_v.at[pl.ds(off_cur, gw)], sem_g).wait()
            pltpu.async_copy(buf_v.at[pl.ds(off_cur, gw)],
                             out_hbm.at[pl.ds(out_w0 + g * gw, gw)], sem_w)
            return carry

        lax.fori_loop(0, _NGROUPS, group, 0)
        pltpu.make_async_copy(buf_v.at[pl.ds(0, gw)],
                              out_hbm.at[pl.ds(out_w0, gw)], sem_w).wait()

    return k(u, idx3d)


def kernel(pokemon_state, move_state, type_state, ability_state, item_state,
           fieldeffect_state, pokemon_table, move_table, type_table,
           ability_table, item_table, fieldeffect_tables):
    B = pokemon_state.shape[0]
    pt, mt, tt, at_, it, ft = _normalize_tables_tc(
        pokemon_table, move_table, type_table, ability_table, item_table,
        fieldeffect_tables.reshape(46, 16))

    u = jnp.concatenate([
        pt.reshape(-1, 16), mt.reshape(-1, 16), tt.reshape(-1, 16),
        at_.reshape(-1, 16), it.reshape(-1, 16), ft,
    ], axis=0)

    raw = jnp.concatenate([
        pokemon_state, move_state, type_state, ability_state, item_state,
        fieldeffect_state], axis=1)
    flat = _QSCALE * jnp.take(raw, _QSLOT, axis=1) + _QBASE
    idx3d = flat.astype(jnp.int32).reshape(_NW, _NGROUPS, _GW)

    out = _sc_gather(u, idx3d)
    return out.reshape(B, _SUBROWS * 16)
